# baseline (speedup 1.0000x reference)
import functools

import jax
import jax.numpy as jnp
from jax import lax
from jax.experimental import pallas as pl
from jax.experimental.pallas import tpu as pltpu
from jax.experimental.pallas import tpu_sc as plsc

N = 10000
E = 320000
D = 128

NC = 2
NS = 16
NW = NC * NS
CH = 128
NCHUNK = 80
EPW = NCHUNK * CH
EPAD = NW * EPW
NPAD = 10240
RPT = NPAD // NS
CPT = NPAD // NS


def _sc_segsum_body(with_cnt, *refs):
    if with_cnt:
        (y_hbm, src_hbm, dst_hbm, zrows_hbm, zcnt_hbm, ones_hbm,
         out_sum, out_cnt,
         acc, idxd_v, sidx0_v, sidx1_v, rows0_v, rows1_v,
         sem0, sem1, isem0, isem1,
         cacc, ones_v, csem) = refs
    else:
        (y_hbm, src_hbm, dst_hbm, zrows_hbm,
         out_sum,
         acc, idxd_v, sidx0_v, sidx1_v, rows0_v, rows1_v,
         sem0, sem1, isem0, isem1) = refs

    c = lax.axis_index("c")
    s = lax.axis_index("s")
    w = c * NS + s

    pltpu.sync_copy(zrows_hbm, acc.at[pl.ds(s * RPT, RPT)])
    if with_cnt:
        pltpu.sync_copy(zcnt_hbm, cacc.at[pl.ds(s * CPT, CPT)])
        pltpu.sync_copy(ones_hbm, ones_v)
    pltpu.sync_copy(dst_hbm.at[w], idxd_v)

    def idx_dma(i, sidx_v, isem):
        return pltpu.make_async_copy(
            src_hbm.at[w, pl.ds(i * CH, CH)], sidx_v, isem)

    def gather(sidx_v, rows_v, sem):
        return pltpu.make_async_copy(y_hbm.at[sidx_v], rows_v, sem)

    def scatter(i, rows_v):
        pltpu.sync_copy(rows_v, acc.at[idxd_v.at[i]], add=True)
        if with_cnt:
            pltpu.async_copy(ones_v, cacc.at[idxd_v.at[i]], csem, add=True)

    plsc.subcore_barrier()

    pltpu.sync_copy(src_hbm.at[w, pl.ds(0, CH)], sidx0_v)
    pltpu.sync_copy(src_hbm.at[w, pl.ds(CH, CH)], sidx1_v)
    gather(sidx0_v, rows0_v, sem0).start()
    gather(sidx1_v, rows1_v, sem1).start()

    def step(j, carry):
        i0 = 2 * j
        gather(sidx0_v, rows0_v, sem0).wait()
        idx_dma(i0 + 2, sidx0_v, isem0).start()
        scatter(i0, rows0_v)
        idx_dma(i0 + 2, sidx0_v, isem0).wait()
        gather(sidx0_v, rows0_v, sem0).start()
        gather(sidx1_v, rows1_v, sem1).wait()
        idx_dma(i0 + 3, sidx1_v, isem1).start()
        scatter(i0 + 1, rows1_v)
        idx_dma(i0 + 3, sidx1_v, isem1).wait()
        gather(sidx1_v, rows1_v, sem1).start()
        return carry

    lax.fori_loop(0, NCHUNK // 2 - 1, step, 0)
    gather(sidx0_v, rows0_v, sem0).wait()
    scatter(NCHUNK - 2, rows0_v)
    gather(sidx1_v, rows1_v, sem1).wait()
    scatter(NCHUNK - 1, rows1_v)
    if with_cnt:
        def drain(i, carry):
            pltpu.make_async_copy(ones_hbm, ones_v, csem).wait()
            return carry
        lax.fori_loop(0, NCHUNK, drain, 0)
    plsc.subcore_barrier()

    pltpu.sync_copy(acc.at[pl.ds(s * RPT, RPT)],
                    out_sum.at[c, pl.ds(s * RPT, RPT)])
    if with_cnt:
        pltpu.sync_copy(cacc.at[pl.ds(s * CPT, CPT)],
                        out_cnt.at[c, pl.ds(s * CPT, CPT)])


def _make_sc_segsum(with_cnt):
    mesh = plsc.VectorSubcoreMesh(core_axis_name="c", subcore_axis_name="s")
    out_type = [jax.ShapeDtypeStruct((NC, NPAD, D), jnp.float32)]
    scratch = [
        pltpu.VMEM_SHARED((NPAD, D), jnp.float32),
        pltpu.VMEM((NCHUNK, CH), jnp.int32),
        pltpu.VMEM((CH,), jnp.int32),
        pltpu.VMEM((CH,), jnp.int32),
        pltpu.VMEM((CH, D), jnp.float32),
        pltpu.VMEM((CH, D), jnp.float32),
        pltpu.SemaphoreType.DMA,
        pltpu.SemaphoreType.DMA,
        pltpu.SemaphoreType.DMA,
        pltpu.SemaphoreType.DMA,
    ]
    if with_cnt:
        out_type.append(jax.ShapeDtypeStruct((NC, NPAD), jnp.float32))
        scratch += [
            pltpu.VMEM_SHARED((NPAD,), jnp.float32),
            pltpu.VMEM((CH,), jnp.float32),
            pltpu.SemaphoreType.DMA,
        ]
    else:
        out_type = out_type[0]
    return pl.kernel(
        functools.partial(_sc_segsum_body, with_cnt),
        out_type=out_type,
        mesh=mesh,
        scratch_types=scratch,
    )


def _tc_layer_body(do_relu, ps_ref, cnt_ref, xin_ref, wl_ref, wr_ref, b_ref,
                   out_ref):
    cnt_pair = cnt_ref[...]
    cnt = cnt_pair[0] + cnt_pair[1]
    rc = 1.0 / jnp.maximum(cnt, 1.0)
    ps = ps_ref[...]
    mean = (ps[0] + ps[1]) * rc
    o = (jnp.dot(mean, wl_ref[...], preferred_element_type=jnp.float32)
         + jnp.dot(xin_ref[...], wr_ref[...],
                   preferred_element_type=jnp.float32)
         + b_ref[...])
    if do_relu:
        o = jnp.maximum(o, 0.0)
    out_ref[...] = o


BM = 1000

_row_blk = pl.BlockSpec((BM, D), lambda i: (i, 0))
_w_blk = pl.BlockSpec((D, D), lambda i: (0, 0))
_b_blk = pl.BlockSpec((1, D), lambda i: (0, 0))
_ps_blk = pl.BlockSpec((NC, BM, D), lambda i: (0, i, 0))
_cnt_blk = pl.BlockSpec((NC, BM, 1), lambda i: (0, i, 0))


def _make_tc_layer(do_relu):
    return pl.pallas_call(
        functools.partial(_tc_layer_body, do_relu),
        grid=(N // BM,),
        in_specs=[_ps_blk, _cnt_blk, _row_blk, _w_blk, _w_blk, _b_blk],
        out_specs=_row_blk,
        out_shape=jax.ShapeDtypeStruct((N, D), jnp.float32),
    )


_tc_layer1 = _make_tc_layer(True)
_tc_layer2 = _make_tc_layer(False)

_sc_segsum_cnt = _make_sc_segsum(True)
_sc_segsum = _make_sc_segsum(False)


def kernel(x, edge_index, W1_l, b1, W1_r, W2_l, b2, W2_r):
    npad_e = EPAD - E
    src_pad = jnp.arange(npad_e, dtype=jnp.int32) % N
    dst_pad = N + (jnp.arange(npad_e, dtype=jnp.int32) % (NPAD - N))
    src = jnp.concatenate([edge_index[0], src_pad]).reshape(NW, EPW)
    dst = jnp.concatenate([edge_index[1], dst_pad]).reshape(NW, NCHUNK, CH)
    zrows = jnp.zeros((RPT, D), jnp.float32)
    zcnt = jnp.zeros((CPT,), jnp.float32)
    ones = jnp.ones((CH,), jnp.float32)
    b1r = b1.reshape(1, D)
    b2r = b2.reshape(1, D)

    psum1, cnt = _sc_segsum_cnt(x, src, dst, zrows, zcnt, ones)
    cnt3 = cnt.reshape(NC, NPAD, 1)
    h = _tc_layer1(psum1, cnt3, x, W1_l, W1_r, b1r)
    psum2 = _sc_segsum(h, src, dst, zrows)
    return _tc_layer2(psum2, cnt3, h, W2_l, W2_r, b2r)


# speedup vs baseline: 1.0103x; 1.0103x over previous
import functools

import jax
import jax.numpy as jnp
from jax import lax
from jax.experimental import pallas as pl
from jax.experimental.pallas import tpu as pltpu
from jax.experimental.pallas import tpu_sc as plsc

N = 10000
E = 320000
D = 128

NC = 2
NS = 16
NW = NC * NS
CH = 128
NCHUNK = 80
EPW = NCHUNK * CH
EPAD = NW * EPW
NPAD = 10240
RPT = NPAD // NS
CPT = NPAD // NS


def _sc_segsum_body(with_cnt, *refs):
    if with_cnt:
        (y_hbm, src_hbm, dst_hbm, zrows_hbm, zcnt_hbm, ones_hbm,
         out_sum, out_cnt,
         acc, idxd_v, sidx0_v, sidx1_v, rows0_v, rows1_v,
         sem0, sem1, isem0, isem1,
         cacc, ones_v, csem) = refs
    else:
        (y_hbm, src_hbm, dst_hbm, zrows_hbm,
         out_sum,
         acc, idxd_v, sidx0_v, sidx1_v, rows0_v, rows1_v,
         sem0, sem1, isem0, isem1) = refs

    c = lax.axis_index("c")
    s = lax.axis_index("s")
    w = c * NS + s

    pltpu.sync_copy(zrows_hbm, acc.at[pl.ds(s * RPT, RPT)])
    if with_cnt:
        pltpu.sync_copy(zcnt_hbm, cacc.at[pl.ds(s * CPT, CPT)])
        pltpu.sync_copy(ones_hbm, ones_v)
    pltpu.sync_copy(dst_hbm.at[w], idxd_v)

    def idx_dma(i, sidx_v, isem):
        return pltpu.make_async_copy(
            src_hbm.at[w, pl.ds(i * CH, CH)], sidx_v, isem)

    def gather(sidx_v, rows_v, sem):
        return pltpu.make_async_copy(y_hbm.at[sidx_v], rows_v, sem)

    def scatter(i, rows_v):
        pltpu.sync_copy(rows_v, acc.at[idxd_v.at[i]], add=True)
        if with_cnt:
            pltpu.async_copy(ones_v, cacc.at[idxd_v.at[i]], csem, add=True)

    plsc.subcore_barrier()

    pltpu.sync_copy(src_hbm.at[w, pl.ds(0, CH)], sidx0_v)
    pltpu.sync_copy(src_hbm.at[w, pl.ds(CH, CH)], sidx1_v)
    gather(sidx0_v, rows0_v, sem0).start()
    gather(sidx1_v, rows1_v, sem1).start()

    def step(j, carry):
        i0 = 2 * j
        gather(sidx0_v, rows0_v, sem0).wait()
        idx_dma(i0 + 2, sidx0_v, isem0).start()
        scatter(i0, rows0_v)
        idx_dma(i0 + 2, sidx0_v, isem0).wait()
        gather(sidx0_v, rows0_v, sem0).start()
        gather(sidx1_v, rows1_v, sem1).wait()
        idx_dma(i0 + 3, sidx1_v, isem1).start()
        scatter(i0 + 1, rows1_v)
        idx_dma(i0 + 3, sidx1_v, isem1).wait()
        gather(sidx1_v, rows1_v, sem1).start()
        return carry

    lax.fori_loop(0, NCHUNK // 2 - 1, step, 0)
    gather(sidx0_v, rows0_v, sem0).wait()
    scatter(NCHUNK - 2, rows0_v)
    gather(sidx1_v, rows1_v, sem1).wait()
    scatter(NCHUNK - 1, rows1_v)
    if with_cnt:
        def drain(i, carry):
            pltpu.make_async_copy(ones_hbm, ones_v, csem).wait()
            return carry
        lax.fori_loop(0, NCHUNK, drain, 0)
    plsc.subcore_barrier()

    pltpu.sync_copy(acc.at[pl.ds(s * RPT, RPT)],
                    out_sum.at[c, pl.ds(s * RPT, RPT)])
    if with_cnt:
        pltpu.sync_copy(cacc.at[pl.ds(s * CPT, CPT)],
                        out_cnt.at[c, pl.ds(s * CPT, CPT)])


def _make_sc_segsum(with_cnt):
    mesh = plsc.VectorSubcoreMesh(core_axis_name="c", subcore_axis_name="s")
    out_type = [jax.ShapeDtypeStruct((NC, NPAD, D), jnp.float32)]
    scratch = [
        pltpu.VMEM_SHARED((NPAD, D), jnp.float32),
        pltpu.VMEM((NCHUNK, CH), jnp.int32),
        pltpu.VMEM((CH,), jnp.int32),
        pltpu.VMEM((CH,), jnp.int32),
        pltpu.VMEM((CH, D), jnp.float32),
        pltpu.VMEM((CH, D), jnp.float32),
        pltpu.SemaphoreType.DMA,
        pltpu.SemaphoreType.DMA,
        pltpu.SemaphoreType.DMA,
        pltpu.SemaphoreType.DMA,
    ]
    if with_cnt:
        out_type.append(jax.ShapeDtypeStruct((NC, NPAD), jnp.float32))
        scratch += [
            pltpu.VMEM_SHARED((NPAD,), jnp.float32),
            pltpu.VMEM((CH,), jnp.float32),
            pltpu.SemaphoreType.DMA,
        ]
    else:
        out_type = out_type[0]
    return pl.kernel(
        functools.partial(_sc_segsum_body, with_cnt),
        out_type=out_type,
        mesh=mesh,
        scratch_types=scratch,
    )


def _tc_layer_body(do_relu, ps_ref, cnt_ref, xin_ref, wcat_ref, b_ref,
                   out_ref):
    cnt_pair = cnt_ref[...]
    cnt = cnt_pair[0] + cnt_pair[1]
    rc = 1.0 / jnp.maximum(cnt, 1.0)
    ps = ps_ref[...]
    mean = (ps[0] + ps[1]) * rc
    mx = jnp.concatenate([mean, xin_ref[...]], axis=1)
    o = (jnp.dot(mx, wcat_ref[...], preferred_element_type=jnp.float32)
         + b_ref[...])
    if do_relu:
        o = jnp.maximum(o, 0.0)
    out_ref[...] = o


BM = 2000

_row_blk = pl.BlockSpec((BM, D), lambda i: (i, 0))
_wcat_blk = pl.BlockSpec((2 * D, D), lambda i: (0, 0))
_b_blk = pl.BlockSpec((1, D), lambda i: (0, 0))
_ps_blk = pl.BlockSpec((NC, BM, D), lambda i: (0, i, 0))
_cnt_blk = pl.BlockSpec((NC, BM, 1), lambda i: (0, i, 0))


def _make_tc_layer(do_relu):
    return pl.pallas_call(
        functools.partial(_tc_layer_body, do_relu),
        grid=(N // BM,),
        in_specs=[_ps_blk, _cnt_blk, _row_blk, _wcat_blk, _b_blk],
        out_specs=_row_blk,
        out_shape=jax.ShapeDtypeStruct((N, D), jnp.float32),
    )


_tc_layer1 = _make_tc_layer(True)
_tc_layer2 = _make_tc_layer(False)

_sc_segsum_cnt = _make_sc_segsum(True)
_sc_segsum = _make_sc_segsum(False)


def kernel(x, edge_index, W1_l, b1, W1_r, W2_l, b2, W2_r):
    npad_e = EPAD - E
    src_pad = jnp.arange(npad_e, dtype=jnp.int32) % N
    dst_pad = N + (jnp.arange(npad_e, dtype=jnp.int32) % (NPAD - N))
    src = jnp.concatenate([edge_index[0], src_pad]).reshape(NW, EPW)
    dst = jnp.concatenate([edge_index[1], dst_pad]).reshape(NW, NCHUNK, CH)
    zrows = jnp.zeros((RPT, D), jnp.float32)
    zcnt = jnp.zeros((CPT,), jnp.float32)
    ones = jnp.ones((CH,), jnp.float32)
    b1r = b1.reshape(1, D)
    b2r = b2.reshape(1, D)
    W1c = jnp.concatenate([W1_l, W1_r], axis=0)
    W2c = jnp.concatenate([W2_l, W2_r], axis=0)

    psum1, cnt = _sc_segsum_cnt(x, src, dst, zrows, zcnt, ones)
    cnt3 = cnt.reshape(NC, NPAD, 1)
    h = _tc_layer1(psum1, cnt3, x, W1c, b1r)
    psum2 = _sc_segsum(h, src, dst, zrows)
    return _tc_layer2(psum2, cnt3, h, W2c, b2r)


# speedup vs baseline: 1.0175x; 1.0071x over previous
import functools

import jax
import jax.numpy as jnp
from jax import lax
from jax.experimental import pallas as pl
from jax.experimental.pallas import tpu as pltpu
from jax.experimental.pallas import tpu_sc as plsc

N = 10000
E = 320000
D = 128

NC = 2
NS = 16
NW = NC * NS
CH = 128
NCHUNK = 80
EPW = NCHUNK * CH
EPAD = NW * EPW
NPAD = 10240
RPT = NPAD // NS
CPT = NPAD // NS


def _sc_segsum_body(with_cnt, *refs):
    if with_cnt:
        (y_hbm, src_hbm, dst_hbm, zrows_hbm, zcnt_hbm, ones_hbm,
         out_sum, out_cnt,
         acc, idxd_v, sidx0_v, sidx1_v, rows0_v, rows1_v,
         sem0, sem1, isem0, isem1,
         cacc, ones_v, csem) = refs
    else:
        (y_hbm, src_hbm, dst_hbm, zrows_hbm,
         out_sum,
         acc, idxd_v, sidx0_v, sidx1_v, rows0_v, rows1_v,
         sem0, sem1, isem0, isem1) = refs

    c = lax.axis_index("c")
    s = lax.axis_index("s")
    w = c * NS + s

    pltpu.sync_copy(zrows_hbm, acc.at[pl.ds(s * RPT, RPT)])
    if with_cnt:
        pltpu.sync_copy(zcnt_hbm, cacc.at[pl.ds(s * CPT, CPT)])
        pltpu.sync_copy(ones_hbm, ones_v)
    pltpu.sync_copy(dst_hbm.at[w], idxd_v)

    def idx_dma(i, sidx_v, isem):
        return pltpu.make_async_copy(
            src_hbm.at[w, pl.ds(i * CH, CH)], sidx_v, isem)

    def gather(sidx_v, rows_v, sem):
        return pltpu.make_async_copy(y_hbm.at[sidx_v], rows_v, sem)

    def scatter(i, rows_v):
        pltpu.sync_copy(rows_v, acc.at[idxd_v.at[i]], add=True)
        if with_cnt:
            pltpu.async_copy(ones_v, cacc.at[idxd_v.at[i]], csem, add=True)

    plsc.subcore_barrier()

    pltpu.sync_copy(src_hbm.at[w, pl.ds(0, CH)], sidx0_v)
    pltpu.sync_copy(src_hbm.at[w, pl.ds(CH, CH)], sidx1_v)
    gather(sidx0_v, rows0_v, sem0).start()
    gather(sidx1_v, rows1_v, sem1).start()

    def step(j, carry):
        i0 = 2 * j
        gather(sidx0_v, rows0_v, sem0).wait()
        idx_dma(i0 + 2, sidx0_v, isem0).start()
        scatter(i0, rows0_v)
        idx_dma(i0 + 2, sidx0_v, isem0).wait()
        gather(sidx0_v, rows0_v, sem0).start()
        gather(sidx1_v, rows1_v, sem1).wait()
        idx_dma(i0 + 3, sidx1_v, isem1).start()
        scatter(i0 + 1, rows1_v)
        idx_dma(i0 + 3, sidx1_v, isem1).wait()
        gather(sidx1_v, rows1_v, sem1).start()
        return carry

    lax.fori_loop(0, NCHUNK // 2 - 1, step, 0)
    gather(sidx0_v, rows0_v, sem0).wait()
    scatter(NCHUNK - 2, rows0_v)
    gather(sidx1_v, rows1_v, sem1).wait()
    scatter(NCHUNK - 1, rows1_v)
    if with_cnt:
        def drain(i, carry):
            pltpu.make_async_copy(ones_hbm, ones_v, csem).wait()
            return carry
        lax.fori_loop(0, NCHUNK, drain, 0)
    plsc.subcore_barrier()

    pltpu.sync_copy(acc.at[pl.ds(s * RPT, RPT)],
                    out_sum.at[c, pl.ds(s * RPT, RPT)])
    if with_cnt:
        pltpu.sync_copy(cacc.at[pl.ds(s * CPT, CPT)],
                        out_cnt.at[c, pl.ds(s * CPT, CPT)])


def _make_sc_segsum(with_cnt):
    mesh = plsc.VectorSubcoreMesh(core_axis_name="c", subcore_axis_name="s")
    out_type = [jax.ShapeDtypeStruct((NC, NPAD, D), jnp.float32)]
    scratch = [
        pltpu.VMEM_SHARED((NPAD, D), jnp.float32),
        pltpu.VMEM((NCHUNK, CH), jnp.int32),
        pltpu.VMEM((CH,), jnp.int32),
        pltpu.VMEM((CH,), jnp.int32),
        pltpu.VMEM((CH, D), jnp.float32),
        pltpu.VMEM((CH, D), jnp.float32),
        pltpu.SemaphoreType.DMA,
        pltpu.SemaphoreType.DMA,
        pltpu.SemaphoreType.DMA,
        pltpu.SemaphoreType.DMA,
    ]
    if with_cnt:
        out_type.append(jax.ShapeDtypeStruct((NC, NPAD), jnp.float32))
        scratch += [
            pltpu.VMEM_SHARED((NPAD,), jnp.float32),
            pltpu.VMEM((CH,), jnp.float32),
            pltpu.SemaphoreType.DMA,
        ]
    else:
        out_type = out_type[0]
    return pl.kernel(
        functools.partial(_sc_segsum_body, with_cnt),
        out_type=out_type,
        mesh=mesh,
        scratch_types=scratch,
    )


def _tc_layer_body(do_relu, ps_ref, cnt_ref, xin_ref, wl_ref, wr_ref, b_ref,
                   out_ref):
    cnt_pair = cnt_ref[...]
    cnt = cnt_pair[0] + cnt_pair[1]
    rc = 1.0 / jnp.maximum(cnt, 1.0)
    ps = ps_ref[...]
    mean = (ps[0] + ps[1]) * rc
    o = (jnp.dot(mean, wl_ref[...], preferred_element_type=jnp.float32)
         + jnp.dot(xin_ref[...], wr_ref[...],
                   preferred_element_type=jnp.float32)
         + b_ref[...])
    if do_relu:
        o = jnp.maximum(o, 0.0)
    out_ref[...] = o


BM = 2000

_row_blk = pl.BlockSpec((BM, D), lambda i: (i, 0))
_w_blk = pl.BlockSpec((D, D), lambda i: (0, 0))
_b_blk = pl.BlockSpec((1, D), lambda i: (0, 0))
_ps_blk = pl.BlockSpec((NC, BM, D), lambda i: (0, i, 0))
_cnt_blk = pl.BlockSpec((NC, BM, 1), lambda i: (0, i, 0))


def _make_tc_layer(do_relu):
    return pl.pallas_call(
        functools.partial(_tc_layer_body, do_relu),
        grid=(N // BM,),
        in_specs=[_ps_blk, _cnt_blk, _row_blk, _w_blk, _w_blk, _b_blk],
        out_specs=_row_blk,
        out_shape=jax.ShapeDtypeStruct((N, D), jnp.float32),
    )


_tc_layer1 = _make_tc_layer(True)
_tc_layer2 = _make_tc_layer(False)

_sc_segsum_cnt = _make_sc_segsum(True)
_sc_segsum = _make_sc_segsum(False)


def kernel(x, edge_index, W1_l, b1, W1_r, W2_l, b2, W2_r):
    npad_e = EPAD - E
    src_pad = jnp.arange(npad_e, dtype=jnp.int32) % N
    dst_pad = N + (jnp.arange(npad_e, dtype=jnp.int32) % (NPAD - N))
    src = jnp.concatenate([edge_index[0], src_pad]).reshape(NW, EPW)
    dst = jnp.concatenate([edge_index[1], dst_pad]).reshape(NW, NCHUNK, CH)
    zrows = jnp.zeros((RPT, D), jnp.float32)
    zcnt = jnp.zeros((CPT,), jnp.float32)
    ones = jnp.ones((CH,), jnp.float32)
    b1r = b1.reshape(1, D)
    b2r = b2.reshape(1, D)

    psum1, cnt = _sc_segsum_cnt(x, src, dst, zrows, zcnt, ones)
    cnt3 = cnt.reshape(NC, NPAD, 1)
    h = _tc_layer1(psum1, cnt3, x, W1_l, W1_r, b1r)
    psum2 = _sc_segsum(h, src, dst, zrows)
    return _tc_layer2(psum2, cnt3, h, W2_l, W2_r, b2r)
